# Initial kernel scaffold; baseline (speedup 1.0000x reference)
import jax
import jax.numpy as jnp
from jax.experimental import pallas as pl


def kernel(x, curr_video_id, W, b):
    raise NotImplementedError("write your pallas kernel here")


# speedup vs baseline: 3.5510x; 3.5510x over previous
import jax
import jax.numpy as jnp
from jax.experimental import pallas as pl
from jax.experimental.pallas import tpu as pltpu

E = 8
D = 768
T = 256


def _moe_dense_body(onehot_ref, x_ref, W_ref, b_ref, out_ref):
    oh = onehot_ref[...]
    acc = jnp.dot(oh, b_ref[...], preferred_element_type=jnp.float32)
    x = x_ref[...]
    for e in range(E):
        m = oh[:, e:e + 1]
        acc = acc + jnp.dot(x * m, W_ref[e], preferred_element_type=jnp.float32)
    out_ref[...] = acc


def kernel(x, curr_video_id, W, b):
    B = x.shape[0]
    eid = curr_video_id.astype(jnp.int32)
    onehot = jax.nn.one_hot(eid, E, dtype=x.dtype)
    num_tiles = B // T

    out = pl.pallas_call(
        _moe_dense_body,
        grid=(num_tiles,),
        in_specs=[
            pl.BlockSpec((T, E), lambda t: (t, 0)),
            pl.BlockSpec((T, D), lambda t: (t, 0)),
            pl.BlockSpec((E, D, D), lambda t: (0, 0, 0)),
            pl.BlockSpec((E, D), lambda t: (0, 0)),
        ],
        out_specs=pl.BlockSpec((T, D), lambda t: (t, 0)),
        out_shape=jax.ShapeDtypeStruct((B, D), x.dtype),
    )(onehot, x, W, b)
    return out
